# baseline (speedup 1.0000x reference)
import functools

import jax
import jax.numpy as jnp
from jax import lax
from jax.experimental import pallas as pl
from jax.experimental.pallas import tpu as pltpu
from jax.experimental.pallas import tpu_sc as plsc

END_TOKEN_VAL = 2

B = 64
V = 100000
S = 2048
L = 16
NC = 2
NS = 16
NW = NC * NS
ROWS_PER_W = B // NW
CH = 50000
NCH = V // CH
CIT = CH // L
NSEG = ROWS_PER_W * NCH
BIG = 2**30


def _greedy_sc(hid_flat, upi_arr, out_ids, flags_i32):
    mesh = plsc.VectorSubcoreMesh(core_axis_name="c", subcore_axis_name="s")

    @functools.partial(
        pl.kernel,
        out_type=[
            jax.ShapeDtypeStruct((B, S), jnp.int32),
            jax.ShapeDtypeStruct((B, L), jnp.int32),
        ],
        mesh=mesh,
        compiler_params=pltpu.CompilerParams(needs_layout_passes=False),
        scratch_types=[
            pltpu.VMEM((CH,), jnp.float32),
            pltpu.VMEM((CH,), jnp.float32),
            pltpu.VMEM((S,), jnp.int32),
            pltpu.VMEM((16,), jnp.int32),
            pltpu.VMEM((16,), jnp.int32),
            pltpu.VMEM((B + L,), jnp.int32),
            pltpu.SemaphoreType.DMA,
            pltpu.SemaphoreType.DMA,
        ],
    )
    def k(hid_hbm, upi_hbm, outids_hbm, flags_hbm, out_hbm, nf_hbm,
          vbuf0, vbuf1, rowbuf, nfbuf, upibuf, flagsbuf, sem0, sem1):
        cid = lax.axis_index("c")
        sid = lax.axis_index("s")
        wid = sid * NC + cid
        row0 = wid * ROWS_PER_W

        pltpu.sync_copy(upi_hbm, upibuf)
        pltpu.sync_copy(flags_hbm, flagsbuf)
        upi = upibuf[...][0]

        vbufs = (vbuf0, vbuf1)
        sems = (sem0, sem1)

        def seg_off(s):
            r, c = divmod(s, NCH)
            return (row0 + r) * V + c * CH

        lane = lax.iota(jnp.int32, L)

        copies = [None] * NSEG
        copies[0] = pltpu.async_copy(
            hid_hbm.at[pl.ds(seg_off(0), CH)], vbufs[0], sems[0])

        m = jnp.full((L,), -jnp.inf, jnp.float32)
        posi = jnp.zeros((L,), jnp.int32)

        for s in range(NSEG):
            r, c = divmod(s, NCH)
            if s + 1 < NSEG:
                copies[s + 1] = pltpu.async_copy(
                    hid_hbm.at[pl.ds(seg_off(s + 1), CH)],
                    vbufs[(s + 1) % 2], sems[(s + 1) % 2])
            copies[s].wait()
            if c == 0:
                m = jnp.full((L,), -jnp.inf, jnp.float32)
                posi = jnp.zeros((L,), jnp.int32)
            buf = vbufs[s % 2]
            base_it = c * CIT

            def body(i, carry, _buf=buf, _base=base_it):
                mm, pp = carry
                v = _buf[pl.ds(i * L, L)]
                upd = v > mm
                mm = jnp.where(upd, v, mm)
                pp = jnp.where(upd, _base + i, pp)
                return mm, pp

            m, posi = lax.fori_loop(0, CIT, body, (m, posi), unroll=8)

            if c == NCH - 1:
                row = row0 + r
                mx = jnp.max(m)
                idxv = posi * L + lane
                cand = jnp.where(m == mx, idxv, BIG)
                p = jnp.min(cand)
                flag = flagsbuf[pl.ds(row, L)][0]
                y = jnp.where(flag != 0, p, jnp.int32(END_TOKEN_VAL))
                pltpu.sync_copy(outids_hbm.at[row], rowbuf)
                plsc.store_scatter(
                    rowbuf, [jnp.full((L,), upi, jnp.int32)],
                    jnp.full((L,), y, jnp.int32))
                pltpu.sync_copy(rowbuf, out_hbm.at[row])
                nfv = jnp.where(y != END_TOKEN_VAL, 1, 0).astype(jnp.int32)
                nfbuf[...] = jnp.full((L,), nfv, jnp.int32)
                pltpu.sync_copy(nfbuf, nf_hbm.at[row])

    return k(hid_flat, upi_arr, out_ids, flags_i32)


def kernel(hidden_state, update_index, out_ids, flags):
    hid_flat = hidden_state.reshape(B * V)
    upi_arr = jnp.full((16,), update_index, jnp.int32)
    flags_i32 = jnp.zeros((B + L,), jnp.int32).at[:B].set(
        flags.reshape(B).astype(jnp.int32))
    out, nf = _greedy_sc(hid_flat, upi_arr, out_ids, flags_i32)
    new_flags = nf[:, :1] != 0
    return out, new_flags


# speedup vs baseline: 6.4409x; 6.4409x over previous
import functools

import jax
import jax.numpy as jnp
from jax import lax
from jax.experimental import pallas as pl
from jax.experimental.pallas import tpu as pltpu
from jax.experimental.pallas import tpu_sc as plsc

END_TOKEN_VAL = 2

B = 64
V = 100000
S = 2048
L = 16
NC = 2
NS = 16
NW = NC * NS
ROWS_PER_W = B // NW
CH0 = 50048
CH1 = 49920
VT = V - CH0 - CH1
SEGS = [(0, CH0), (CH0, CH1)]
NCH = len(SEGS)
NSEG = ROWS_PER_W * NCH
BIG = 2**30


def _greedy_sc(hid, tails, upi_arr, out_ids, flags_i32):
    mesh = plsc.VectorSubcoreMesh(core_axis_name="c", subcore_axis_name="s")

    @functools.partial(
        pl.kernel,
        out_type=[
            jax.ShapeDtypeStruct((B, S), jnp.int32),
            jax.ShapeDtypeStruct((B, L), jnp.int32),
        ],
        mesh=mesh,
        compiler_params=pltpu.CompilerParams(needs_layout_passes=False),
        scratch_types=[
            pltpu.VMEM((CH0,), jnp.float32),
            pltpu.VMEM((CH0,), jnp.float32),
            pltpu.VMEM((VT,), jnp.float32),
            pltpu.VMEM((S,), jnp.int32),
            pltpu.VMEM((16,), jnp.int32),
            pltpu.VMEM((16,), jnp.int32),
            pltpu.VMEM((B + L,), jnp.int32),
            pltpu.SemaphoreType.DMA,
            pltpu.SemaphoreType.DMA,
        ],
    )
    def k(hid_hbm, tails_hbm, upi_hbm, outids_hbm, flags_hbm, out_hbm, nf_hbm,
          vbuf0, vbuf1, tailbuf, rowbuf, nfbuf, upibuf, flagsbuf, sem0, sem1):
        cid = lax.axis_index("c")
        sid = lax.axis_index("s")
        wid = sid * NC + cid
        row0 = wid * ROWS_PER_W

        pltpu.sync_copy(upi_hbm, upibuf)
        pltpu.sync_copy(flags_hbm, flagsbuf)
        upi = upibuf[...][0]

        vbufs = (vbuf0, vbuf1)
        sems = (sem0, sem1)

        def seg_src(s):
            r, c = divmod(s, NCH)
            off, sz = SEGS[c]
            return hid_hbm.at[row0 + r, 0, pl.ds(off, sz)]

        lane = lax.iota(jnp.int32, L)

        copies = [None] * NSEG
        copies[0] = pltpu.async_copy(seg_src(0), vbufs[0].at[pl.ds(0, CH0)],
                                     sems[0])

        m = jnp.full((L,), -jnp.inf, jnp.float32)
        posi = jnp.zeros((L,), jnp.int32)

        for s in range(NSEG):
            r, c = divmod(s, NCH)
            off, sz = SEGS[c]
            if s + 1 < NSEG:
                nsz = SEGS[(s + 1) % NCH][1]
                copies[s + 1] = pltpu.async_copy(
                    seg_src(s + 1), vbufs[(s + 1) % 2].at[pl.ds(0, nsz)],
                    sems[(s + 1) % 2])
            copies[s].wait()
            if c == 0:
                m = jnp.full((L,), -jnp.inf, jnp.float32)
                posi = jnp.zeros((L,), jnp.int32)
            buf = vbufs[s % 2]
            base_it = off // L

            def body(i, carry, _buf=buf, _base=base_it):
                mm, pp = carry
                v = _buf[pl.ds(i * L, L)]
                upd = v > mm
                mm = jnp.where(upd, v, mm)
                pp = jnp.where(upd, _base + i, pp)
                return mm, pp

            m, posi = lax.fori_loop(0, sz // L, body, (m, posi), unroll=8)

            if c == NCH - 1:
                row = row0 + r
                pltpu.sync_copy(tails_hbm.at[pl.ds(row * VT, VT)], tailbuf)
                tbase = (off + sz) // L
                for j in range(VT // L):
                    v = tailbuf[pl.ds(j * L, L)]
                    upd = v > m
                    m = jnp.where(upd, v, m)
                    posi = jnp.where(upd, tbase + j, posi)
                mx = jnp.max(m)
                idxv = posi * L + lane
                cand = jnp.where(m == mx, idxv, BIG)
                p = jnp.min(cand)
                flag = flagsbuf[pl.ds(row, L)][0]
                y = jnp.where(flag != 0, p, jnp.int32(END_TOKEN_VAL))
                pltpu.sync_copy(outids_hbm.at[row], rowbuf)
                plsc.store_scatter(
                    rowbuf, [jnp.full((L,), upi, jnp.int32)],
                    jnp.full((L,), y, jnp.int32))
                pltpu.sync_copy(rowbuf, out_hbm.at[row])
                nfv = jnp.where(y != END_TOKEN_VAL, 1, 0).astype(jnp.int32)
                nfbuf[...] = jnp.full((L,), nfv, jnp.int32)
                pltpu.sync_copy(nfbuf, nf_hbm.at[row])

    return k(hid, tails, upi_arr, out_ids, flags_i32)


def kernel(hidden_state, update_index, out_ids, flags):
    tails = hidden_state[:, 0, CH0 + CH1:].reshape(B * VT)
    upi_arr = jnp.full((16,), update_index, jnp.int32)
    flags_i32 = jnp.zeros((B + L,), jnp.int32).at[:B].set(
        flags.reshape(B).astype(jnp.int32))
    out, nf = _greedy_sc(hidden_state, tails, upi_arr, out_ids, flags_i32)
    new_flags = nf[:, :1] != 0
    return out, new_flags


# speedup vs baseline: 7.0666x; 1.0971x over previous
import functools

import jax
import jax.numpy as jnp
from jax import lax
from jax.experimental import pallas as pl
from jax.experimental.pallas import tpu as pltpu
from jax.experimental.pallas import tpu_sc as plsc

END_TOKEN_VAL = 2

B = 64
V = 100000
S = 2048
L = 16
NC = 2
NS = 16
NW = NC * NS
ROWS_PER_W = B // NW
CH0 = 50048
CH1 = 49920
VT = V - CH0 - CH1
SEGS = [(0, CH0), (CH0, CH1)]
NCH = len(SEGS)
NSEG = ROWS_PER_W * NCH
KACC = 4
BIG = 2**30


def _greedy_sc(hid, tails, upi_arr, out_ids, flags_i32):
    mesh = plsc.VectorSubcoreMesh(core_axis_name="c", subcore_axis_name="s")

    @functools.partial(
        pl.kernel,
        out_type=[
            jax.ShapeDtypeStruct((B, S), jnp.int32),
            jax.ShapeDtypeStruct((B, L), jnp.int32),
        ],
        mesh=mesh,
        compiler_params=pltpu.CompilerParams(needs_layout_passes=False),
        scratch_types=[
            pltpu.VMEM((CH0,), jnp.float32),
            pltpu.VMEM((CH0,), jnp.float32),
            pltpu.VMEM((VT,), jnp.float32),
            pltpu.VMEM((S,), jnp.int32),
            pltpu.VMEM((16,), jnp.int32),
            pltpu.VMEM((16,), jnp.int32),
            pltpu.VMEM((B + L,), jnp.int32),
            pltpu.SemaphoreType.DMA,
            pltpu.SemaphoreType.DMA,
        ],
    )
    def k(hid_hbm, tails_hbm, upi_hbm, outids_hbm, flags_hbm, out_hbm, nf_hbm,
          vbuf0, vbuf1, tailbuf, rowbuf, nfbuf, upibuf, flagsbuf, sem0, sem1):
        cid = lax.axis_index("c")
        sid = lax.axis_index("s")
        wid = sid * NC + cid
        row0 = wid * ROWS_PER_W

        pltpu.sync_copy(upi_hbm, upibuf)
        pltpu.sync_copy(flags_hbm, flagsbuf)
        upi = upibuf[...][0]

        vbufs = (vbuf0, vbuf1)
        sems = (sem0, sem1)

        def seg_src(s):
            r, c = divmod(s, NCH)
            off, sz = SEGS[c]
            return hid_hbm.at[row0 + r, 0, pl.ds(off, sz)]

        lane = lax.iota(jnp.int32, L)
        neginf = jnp.full((L,), -jnp.inf, jnp.float32)
        zeros = jnp.zeros((L,), jnp.int32)

        copies = [None] * NSEG
        copies[0] = pltpu.async_copy(seg_src(0), vbufs[0].at[pl.ds(0, CH0)],
                                     sems[0])

        accs = None

        for s in range(NSEG):
            r, c = divmod(s, NCH)
            off, sz = SEGS[c]
            if s + 1 < NSEG:
                nsz = SEGS[(s + 1) % NCH][1]
                copies[s + 1] = pltpu.async_copy(
                    seg_src(s + 1), vbufs[(s + 1) % 2].at[pl.ds(0, nsz)],
                    sems[(s + 1) % 2])
            copies[s].wait()
            if c == 0:
                accs = [(neginf, zeros) for _ in range(KACC)]
            buf = vbufs[s % 2]

            idxs = [off + k * L + lane for k in range(KACC)]

            def body(i, carry, _buf=buf):
                st = list(carry)
                for k in range(KACC):
                    mm, pp = st[2 * k], st[2 * k + 1]
                    ix = st[2 * KACC + k]
                    v = _buf[pl.ds(i * (KACC * L) + k * L, L)]
                    upd = v > mm
                    st[2 * k] = jnp.where(upd, v, mm)
                    st[2 * k + 1] = jnp.where(upd, ix, pp)
                    st[2 * KACC + k] = ix + KACC * L
                return tuple(st)

            flat = tuple(x for a in accs for x in a) + tuple(idxs)
            flat = lax.fori_loop(0, sz // (KACC * L), body, flat, unroll=4)
            accs = [(flat[2 * k], flat[2 * k + 1]) for k in range(KACC)]

            if c == NCH - 1:
                row = row0 + r
                m, posi = accs[0]
                for mm, pp in accs[1:]:
                    take = (mm > m) | ((mm == m) & (pp < posi))
                    m = jnp.where(take, mm, m)
                    posi = jnp.where(take, pp, posi)
                pltpu.sync_copy(tails_hbm.at[pl.ds(row * VT, VT)], tailbuf)
                for j in range(VT // L):
                    v = tailbuf[pl.ds(j * L, L)]
                    ix = (off + sz + j * L) + lane
                    upd = v > m
                    m = jnp.where(upd, v, m)
                    posi = jnp.where(upd, ix, posi)
                mx = jnp.max(m)
                cand = jnp.where(m == mx, posi, BIG)
                p = jnp.min(cand)
                flag = flagsbuf[pl.ds(row, L)][0]
                y = jnp.where(flag != 0, p, jnp.int32(END_TOKEN_VAL))
                pltpu.sync_copy(outids_hbm.at[row], rowbuf)
                plsc.store_scatter(
                    rowbuf, [jnp.full((L,), upi, jnp.int32)],
                    jnp.full((L,), y, jnp.int32))
                pltpu.sync_copy(rowbuf, out_hbm.at[row])
                nfv = jnp.where(y != END_TOKEN_VAL, 1, 0).astype(jnp.int32)
                nfbuf[...] = jnp.full((L,), nfv, jnp.int32)
                pltpu.sync_copy(nfbuf, nf_hbm.at[row])

    return k(hid, tails, upi_arr, out_ids, flags_i32)


def kernel(hidden_state, update_index, out_ids, flags):
    tails = hidden_state[:, 0, CH0 + CH1:].reshape(B * VT)
    upi_arr = jnp.full((16,), update_index, jnp.int32)
    flags_i32 = jnp.zeros((B + L,), jnp.int32).at[:B].set(
        flags.reshape(B).astype(jnp.int32))
    out, nf = _greedy_sc(hidden_state, tails, upi_arr, out_ids, flags_i32)
    new_flags = nf[:, :1] != 0
    return out, new_flags


# speedup vs baseline: 7.5385x; 1.0668x over previous
import functools

import jax
import jax.numpy as jnp
from jax import lax
from jax.experimental import pallas as pl
from jax.experimental.pallas import tpu as pltpu
from jax.experimental.pallas import tpu_sc as plsc

END_TOKEN_VAL = 2

B = 64
BSC = 32
BTC = B - BSC
V = 100000
S = 2048
L = 16
NC = 2
NS = 16
NW = NC * NS
CH0 = 50048
CH1 = 49920
VT = V - CH0 - CH1
SEGS = [(0, CH0), (CH0, CH1)]
NSEG = len(SEGS)
KACC = 4
BIG = 2**30

TBLK = 12800
TNB = -(-V // TBLK)


def _greedy_sc(hid, tails, upi_arr, out_ids, flags_i32):
    mesh = plsc.VectorSubcoreMesh(core_axis_name="c", subcore_axis_name="s")

    @functools.partial(
        pl.kernel,
        out_type=[
            jax.ShapeDtypeStruct((BSC, S), jnp.int32),
            jax.ShapeDtypeStruct((BSC, L), jnp.int32),
        ],
        mesh=mesh,
        compiler_params=pltpu.CompilerParams(needs_layout_passes=False),
        scratch_types=[
            pltpu.VMEM((CH0,), jnp.float32),
            pltpu.VMEM((CH0,), jnp.float32),
            pltpu.VMEM((VT,), jnp.float32),
            pltpu.VMEM((S,), jnp.int32),
            pltpu.VMEM((16,), jnp.int32),
            pltpu.VMEM((16,), jnp.int32),
            pltpu.VMEM((B + L,), jnp.int32),
            pltpu.SemaphoreType.DMA,
            pltpu.SemaphoreType.DMA,
        ],
    )
    def k(hid_hbm, tails_hbm, upi_hbm, outids_hbm, flags_hbm, out_hbm, nf_hbm,
          vbuf0, vbuf1, tailbuf, rowbuf, nfbuf, upibuf, flagsbuf, sem0, sem1):
        cid = lax.axis_index("c")
        sid = lax.axis_index("s")
        row = sid * NC + cid

        pltpu.sync_copy(upi_hbm, upibuf)
        pltpu.sync_copy(flags_hbm, flagsbuf)
        upi = upibuf[...][0]

        vbufs = (vbuf0, vbuf1)
        sems = (sem0, sem1)

        def seg_src(c):
            off, sz = SEGS[c]
            return hid_hbm.at[row, 0, pl.ds(off, sz)]

        lane = lax.iota(jnp.int32, L)
        neginf = jnp.full((L,), -jnp.inf, jnp.float32)
        zeros = jnp.zeros((L,), jnp.int32)

        copies = [None] * NSEG
        copies[0] = pltpu.async_copy(seg_src(0), vbufs[0].at[pl.ds(0, CH0)],
                                     sems[0])
        accs = [(neginf, zeros) for _ in range(KACC)]

        for c in range(NSEG):
            off, sz = SEGS[c]
            if c + 1 < NSEG:
                nsz = SEGS[c + 1][1]
                copies[c + 1] = pltpu.async_copy(
                    seg_src(c + 1), vbufs[(c + 1) % 2].at[pl.ds(0, nsz)],
                    sems[(c + 1) % 2])
            copies[c].wait()
            buf = vbufs[c % 2]
            idxs = [off + k * L + lane for k in range(KACC)]

            def body(i, carry, _buf=buf):
                st = list(carry)
                for k in range(KACC):
                    mm, pp = st[2 * k], st[2 * k + 1]
                    ix = st[2 * KACC + k]
                    v = _buf[pl.ds(i * (KACC * L) + k * L, L)]
                    upd = v > mm
                    st[2 * k] = jnp.where(upd, v, mm)
                    st[2 * k + 1] = jnp.where(upd, ix, pp)
                    st[2 * KACC + k] = ix + KACC * L
                return tuple(st)

            flat = tuple(x for a in accs for x in a) + tuple(idxs)
            flat = lax.fori_loop(0, sz // (KACC * L), body, flat, unroll=4)
            accs = [(flat[2 * k], flat[2 * k + 1]) for k in range(KACC)]

        m, posi = accs[0]
        for mm, pp in accs[1:]:
            take = (mm > m) | ((mm == m) & (pp < posi))
            m = jnp.where(take, mm, m)
            posi = jnp.where(take, pp, posi)
        pltpu.sync_copy(tails_hbm.at[pl.ds(row * VT, VT)], tailbuf)
        for j in range(VT // L):
            v = tailbuf[pl.ds(j * L, L)]
            ix = (CH0 + CH1 + j * L) + lane
            upd = v > m
            m = jnp.where(upd, v, m)
            posi = jnp.where(upd, ix, posi)
        mx = jnp.max(m)
        cand = jnp.where(m == mx, posi, BIG)
        p = jnp.min(cand)
        flag = flagsbuf[pl.ds(row, L)][0]
        y = jnp.where(flag != 0, p, jnp.int32(END_TOKEN_VAL))
        pltpu.sync_copy(outids_hbm.at[row], rowbuf)
        plsc.store_scatter(
            rowbuf, [jnp.full((L,), upi, jnp.int32)],
            jnp.full((L,), y, jnp.int32))
        pltpu.sync_copy(rowbuf, out_hbm.at[row])
        nfv = jnp.where(y != END_TOKEN_VAL, 1, 0).astype(jnp.int32)
        nfbuf[...] = jnp.full((L,), nfv, jnp.int32)
        pltpu.sync_copy(nfbuf, nf_hbm.at[row])

    return k(hid, tails, upi_arr, out_ids, flags_i32)


def _tc_body(upi_ref, hid_ref, outids_ref, flags_ref, out_ref, y_ref,
             macc_ref, iacc_ref):
    j = pl.program_id(0)
    x = hid_ref[...].reshape(BTC, TBLK)
    colidx = (j * TBLK
              + jax.lax.broadcasted_iota(jnp.int32, (BTC, TBLK), 1))
    x = jnp.where(colidx < V, x, -jnp.inf)

    @pl.when(j == 0)
    def _():
        macc_ref[...] = jnp.full((BTC, 128), -jnp.inf, jnp.float32)
        iacc_ref[...] = jnp.zeros((BTC, 128), jnp.int32)

    mac = macc_ref[...]
    iac = iacc_ref[...]
    for k in range(TBLK // 128):
        v = x[:, k * 128:(k + 1) * 128]
        ci = colidx[:, k * 128:(k + 1) * 128]
        upd = v > mac
        mac = jnp.where(upd, v, mac)
        iac = jnp.where(upd, ci, iac)
    macc_ref[...] = mac
    iacc_ref[...] = iac

    @pl.when(j == TNB - 1)
    def _():
        rowmax = jnp.max(mac, axis=1, keepdims=True)
        cand = jnp.where(mac == rowmax, iac, BIG)
        p = jnp.min(cand, axis=1, keepdims=True)
        flag = flags_ref[...]
        y = jnp.where(flag != 0, p, END_TOKEN_VAL).astype(jnp.int32)
        upi = upi_ref[0]
        cols = jax.lax.broadcasted_iota(jnp.int32, (BTC, S), 1)
        out_ref[...] = jnp.where(cols == upi, y, outids_ref[...])
        y_ref[...] = jnp.broadcast_to(y, (BTC, 128))


def _greedy_tc(hid, upi_arr, out_ids, flags_i32):
    grid_spec = pltpu.PrefetchScalarGridSpec(
        num_scalar_prefetch=1,
        grid=(TNB,),
        in_specs=[
            pl.BlockSpec((BTC, 1, TBLK), lambda j, upi: (1, 0, j)),
            pl.BlockSpec((BTC, S), lambda j, upi: (1, 0)),
            pl.BlockSpec((BTC, 1), lambda j, upi: (0, 0)),
        ],
        out_specs=[
            pl.BlockSpec((BTC, S), lambda j, upi: (0, 0)),
            pl.BlockSpec((BTC, 128), lambda j, upi: (0, 0)),
        ],
        scratch_shapes=[
            pltpu.VMEM((BTC, 128), jnp.float32),
            pltpu.VMEM((BTC, 128), jnp.int32),
        ],
    )
    return pl.pallas_call(
        _tc_body,
        grid_spec=grid_spec,
        out_shape=[
            jax.ShapeDtypeStruct((BTC, S), jnp.int32),
            jax.ShapeDtypeStruct((BTC, 128), jnp.int32),
        ],
    )(upi_arr[:1], hid, out_ids, flags_i32)


def kernel(hidden_state, update_index, out_ids, flags):
    tails = hidden_state[:BSC, 0, CH0 + CH1:].reshape(BSC * VT)
    upi_arr = jnp.full((16,), update_index, jnp.int32)
    flags_flat = flags.reshape(B).astype(jnp.int32)
    flags_pad = jnp.zeros((B + L,), jnp.int32).at[:B].set(flags_flat)
    out_sc, nf_sc = _greedy_sc(hidden_state, tails, upi_arr, out_ids,
                               flags_pad)
    out_tc, y_tc = _greedy_tc(hidden_state, upi_arr, out_ids,
                              flags_flat[BSC:].reshape(BTC, 1))
    out = jnp.concatenate([out_sc, out_tc], axis=0)
    new_flags = jnp.concatenate(
        [nf_sc[:, :1] != 0, y_tc[:, :1] != END_TOKEN_VAL], axis=0)
    return out, new_flags


# speedup vs baseline: 7.8059x; 1.0355x over previous
import functools

import jax
import jax.numpy as jnp
from jax import lax
from jax.experimental import pallas as pl
from jax.experimental.pallas import tpu as pltpu
from jax.experimental.pallas import tpu_sc as plsc

END_TOKEN_VAL = 2

B = 64
BSC = 32
BTC = B - BSC
V = 100000
S = 2048
L = 16
NC = 2
NS = 16
NW = NC * NS
_SEG_SZ = [25088, 25088, 24960, 24832]
SEGS = []
_off = 0
for _sz in _SEG_SZ:
    SEGS.append((_off, _sz))
    _off += _sz
VT = V - _off
CHMAX = max(_SEG_SZ)
NSEG = len(SEGS)
KACC = 4
BIG = 2**30

TBLK = 12800
TNB = -(-V // TBLK)


def _greedy_sc(hid, tails, upi_arr, out_ids, flags_i32):
    mesh = plsc.VectorSubcoreMesh(core_axis_name="c", subcore_axis_name="s")

    @functools.partial(
        pl.kernel,
        out_type=[
            jax.ShapeDtypeStruct((BSC, S), jnp.int32),
            jax.ShapeDtypeStruct((BSC, L), jnp.int32),
        ],
        mesh=mesh,
        compiler_params=pltpu.CompilerParams(needs_layout_passes=False),
        scratch_types=(
            [pltpu.VMEM((CHMAX,), jnp.float32) for _ in range(NSEG)]
            + [
                pltpu.VMEM((VT,), jnp.float32),
                pltpu.VMEM((S,), jnp.int32),
                pltpu.VMEM((16,), jnp.int32),
                pltpu.VMEM((16,), jnp.int32),
                pltpu.VMEM((B + L,), jnp.int32),
            ]
            + [pltpu.SemaphoreType.DMA for _ in range(NSEG)]
        ),
    )
    def k(hid_hbm, tails_hbm, upi_hbm, outids_hbm, flags_hbm, out_hbm, nf_hbm,
          *rest):
        vbufs = rest[:NSEG]
        tailbuf, rowbuf, nfbuf, upibuf, flagsbuf = rest[NSEG:NSEG + 5]
        sems = rest[NSEG + 5:]
        cid = lax.axis_index("c")
        sid = lax.axis_index("s")
        row = sid * NC + cid

        def seg_src(c):
            off, sz = SEGS[c]
            return hid_hbm.at[row, 0, pl.ds(off, sz)]

        lane = lax.iota(jnp.int32, L)
        neginf = jnp.full((L,), -jnp.inf, jnp.float32)
        zeros = jnp.zeros((L,), jnp.int32)

        copies = [
            pltpu.async_copy(seg_src(c), vbufs[c].at[pl.ds(0, SEGS[c][1])],
                             sems[c])
            for c in range(NSEG)
        ]
        pltpu.sync_copy(upi_hbm, upibuf)
        pltpu.sync_copy(flags_hbm, flagsbuf)
        upi = upibuf[...][0]

        accs = [(neginf, zeros) for _ in range(KACC)]

        for c in range(NSEG):
            off, sz = SEGS[c]
            copies[c].wait()
            buf = vbufs[c]
            idxs = [off + k * L + lane for k in range(KACC)]

            def body(i, carry, _buf=buf):
                st = list(carry)
                for k in range(KACC):
                    mm, pp = st[2 * k], st[2 * k + 1]
                    ix = st[2 * KACC + k]
                    v = _buf[pl.ds(i * (KACC * L) + k * L, L)]
                    upd = v > mm
                    st[2 * k] = jnp.where(upd, v, mm)
                    st[2 * k + 1] = jnp.where(upd, ix, pp)
                    st[2 * KACC + k] = ix + KACC * L
                return tuple(st)

            flat = tuple(x for a in accs for x in a) + tuple(idxs)
            flat = lax.fori_loop(0, sz // (KACC * L), body, flat, unroll=4)
            accs = [(flat[2 * k], flat[2 * k + 1]) for k in range(KACC)]

        m, posi = accs[0]
        for mm, pp in accs[1:]:
            take = (mm > m) | ((mm == m) & (pp < posi))
            m = jnp.where(take, mm, m)
            posi = jnp.where(take, pp, posi)
        pltpu.sync_copy(tails_hbm.at[pl.ds(row * VT, VT)], tailbuf)
        for j in range(VT // L):
            v = tailbuf[pl.ds(j * L, L)]
            ix = (V - VT + j * L) + lane
            upd = v > m
            m = jnp.where(upd, v, m)
            posi = jnp.where(upd, ix, posi)
        mx = jnp.max(m)
        cand = jnp.where(m == mx, posi, BIG)
        p = jnp.min(cand)
        flag = flagsbuf[pl.ds(row, L)][0]
        y = jnp.where(flag != 0, p, jnp.int32(END_TOKEN_VAL))
        pltpu.sync_copy(outids_hbm.at[row], rowbuf)
        plsc.store_scatter(
            rowbuf, [jnp.full((L,), upi, jnp.int32)],
            jnp.full((L,), y, jnp.int32))
        pltpu.sync_copy(rowbuf, out_hbm.at[row])
        nfv = jnp.where(y != END_TOKEN_VAL, 1, 0).astype(jnp.int32)
        nfbuf[...] = jnp.full((L,), nfv, jnp.int32)
        pltpu.sync_copy(nfbuf, nf_hbm.at[row])

    return k(hid, tails, upi_arr, out_ids, flags_i32)


def _tc_body(upi_ref, hid_ref, outids_ref, flags_ref, out_ref, y_ref,
             macc_ref, iacc_ref):
    j = pl.program_id(0)
    x = hid_ref[...].reshape(BTC, TBLK)
    colidx = (j * TBLK
              + jax.lax.broadcasted_iota(jnp.int32, (BTC, TBLK), 1))
    x = jnp.where(colidx < V, x, -jnp.inf)

    @pl.when(j == 0)
    def _():
        macc_ref[...] = jnp.full((BTC, 128), -jnp.inf, jnp.float32)
        iacc_ref[...] = jnp.zeros((BTC, 128), jnp.int32)

    mac = macc_ref[...]
    iac = iacc_ref[...]
    for k in range(TBLK // 128):
        v = x[:, k * 128:(k + 1) * 128]
        ci = colidx[:, k * 128:(k + 1) * 128]
        upd = v > mac
        mac = jnp.where(upd, v, mac)
        iac = jnp.where(upd, ci, iac)
    macc_ref[...] = mac
    iacc_ref[...] = iac

    @pl.when(j == TNB - 1)
    def _():
        rowmax = jnp.max(mac, axis=1, keepdims=True)
        cand = jnp.where(mac == rowmax, iac, BIG)
        p = jnp.min(cand, axis=1, keepdims=True)
        flag = flags_ref[...]
        y = jnp.where(flag != 0, p, END_TOKEN_VAL).astype(jnp.int32)
        upi = upi_ref[0]
        cols = jax.lax.broadcasted_iota(jnp.int32, (BTC, S), 1)
        out_ref[...] = jnp.where(cols == upi, y, outids_ref[...])
        y_ref[...] = jnp.broadcast_to(y, (BTC, 128))


def _greedy_tc(hid, upi_arr, out_ids, flags_i32):
    grid_spec = pltpu.PrefetchScalarGridSpec(
        num_scalar_prefetch=1,
        grid=(TNB,),
        in_specs=[
            pl.BlockSpec((BTC, 1, TBLK), lambda j, upi: (1, 0, j)),
            pl.BlockSpec((BTC, S), lambda j, upi: (1, 0)),
            pl.BlockSpec((BTC, 1), lambda j, upi: (0, 0)),
        ],
        out_specs=[
            pl.BlockSpec((BTC, S), lambda j, upi: (0, 0)),
            pl.BlockSpec((BTC, 128), lambda j, upi: (0, 0)),
        ],
        scratch_shapes=[
            pltpu.VMEM((BTC, 128), jnp.float32),
            pltpu.VMEM((BTC, 128), jnp.int32),
        ],
    )
    return pl.pallas_call(
        _tc_body,
        grid_spec=grid_spec,
        out_shape=[
            jax.ShapeDtypeStruct((BTC, S), jnp.int32),
            jax.ShapeDtypeStruct((BTC, 128), jnp.int32),
        ],
    )(upi_arr[:1], hid, out_ids, flags_i32)


def kernel(hidden_state, update_index, out_ids, flags):
    tails = hidden_state[:BSC, 0, V - VT:].reshape(BSC * VT)
    upi_arr = jnp.full((16,), update_index, jnp.int32)
    flags_flat = flags.reshape(B).astype(jnp.int32)
    flags_pad = jnp.zeros((B + L,), jnp.int32).at[:B].set(flags_flat)
    out_sc, nf_sc = _greedy_sc(hidden_state, tails, upi_arr, out_ids,
                               flags_pad)
    out_tc, y_tc = _greedy_tc(hidden_state, upi_arr, out_ids,
                              flags_flat[BSC:].reshape(BTC, 1))
    out = jnp.concatenate([out_sc, out_tc], axis=0)
    new_flags = jnp.concatenate(
        [nf_sc[:, :1] != 0, y_tc[:, :1] != END_TOKEN_VAL], axis=0)
    return out, new_flags


# speedup vs baseline: 7.9614x; 1.0199x over previous
import functools

import jax
import jax.numpy as jnp
from jax import lax
from jax.experimental import pallas as pl
from jax.experimental.pallas import tpu as pltpu
from jax.experimental.pallas import tpu_sc as plsc

END_TOKEN_VAL = 2

B = 64
BSC = 32
BTC = B - BSC
V = 100000
S = 2048
L = 16
NC = 2
NS = 16
NW = NC * NS
_SEG_SZ = [25088, 25088, 24960, 24832]
SEGS = []
_off = 0
for _sz in _SEG_SZ:
    SEGS.append((_off, _sz))
    _off += _sz
VT = V - _off
CHMAX = max(_SEG_SZ)
NSEG = len(SEGS)
KACC = 4
BIG = 2**30

AUX_FLAGS = 16
AUX_TAILS = AUX_FLAGS + BSC + L
AUX_LEN = AUX_TAILS + BSC * VT

TBLK = 12800
TNB = -(-V // TBLK)


def _greedy_sc(hid, aux):
    mesh = plsc.VectorSubcoreMesh(core_axis_name="c", subcore_axis_name="s")

    @functools.partial(
        pl.kernel,
        out_type=jax.ShapeDtypeStruct((BSC, L), jnp.int32),
        mesh=mesh,
        compiler_params=pltpu.CompilerParams(needs_layout_passes=False),
        scratch_types=(
            [pltpu.VMEM((CHMAX,), jnp.float32) for _ in range(NSEG)]
            + [
                pltpu.VMEM((AUX_LEN,), jnp.int32),
                pltpu.VMEM((16,), jnp.int32),
            ]
            + [pltpu.SemaphoreType.DMA for _ in range(NSEG)]
        ),
    )
    def k(hid_hbm, aux_hbm, y_hbm, *rest):
        vbufs = rest[:NSEG]
        auxbuf, ybuf = rest[NSEG:NSEG + 2]
        sems = rest[NSEG + 2:]
        cid = lax.axis_index("c")
        sid = lax.axis_index("s")
        row = sid * NC + cid

        def seg_src(c):
            off, sz = SEGS[c]
            return hid_hbm.at[row, 0, pl.ds(off, sz)]

        lane = lax.iota(jnp.int32, L)
        neginf = jnp.full((L,), -jnp.inf, jnp.float32)
        zeros = jnp.zeros((L,), jnp.int32)

        copies = [
            pltpu.async_copy(seg_src(c), vbufs[c].at[pl.ds(0, SEGS[c][1])],
                             sems[c])
            for c in range(NSEG)
        ]
        pltpu.sync_copy(aux_hbm, auxbuf)
        upi = auxbuf[pl.ds(0, L)][0]

        accs = [(neginf, zeros) for _ in range(KACC)]

        for c in range(NSEG):
            off, sz = SEGS[c]
            copies[c].wait()
            buf = vbufs[c]
            idxs = [off + k * L + lane for k in range(KACC)]

            def body(i, carry, _buf=buf):
                st = list(carry)
                for k in range(KACC):
                    mm, pp = st[2 * k], st[2 * k + 1]
                    ix = st[2 * KACC + k]
                    v = _buf[pl.ds(i * (KACC * L) + k * L, L)]
                    upd = v > mm
                    st[2 * k] = jnp.where(upd, v, mm)
                    st[2 * k + 1] = jnp.where(upd, ix, pp)
                    st[2 * KACC + k] = ix + KACC * L
                return tuple(st)

            flat = tuple(x for a in accs for x in a) + tuple(idxs)
            flat = lax.fori_loop(0, sz // (KACC * L), body, flat, unroll=4)
            accs = [(flat[2 * k], flat[2 * k + 1]) for k in range(KACC)]

        m, posi = accs[0]
        for mm, pp in accs[1:]:
            take = (mm > m) | ((mm == m) & (pp < posi))
            m = jnp.where(take, mm, m)
            posi = jnp.where(take, pp, posi)
        for j in range(VT // L):
            v = plsc.bitcast(
                auxbuf[pl.ds(AUX_TAILS + row * VT + j * L, L)], jnp.float32)
            ix = (V - VT + j * L) + lane
            upd = v > m
            m = jnp.where(upd, v, m)
            posi = jnp.where(upd, ix, posi)
        mx = jnp.max(m)
        cand = jnp.where(m == mx, posi, BIG)
        p = jnp.min(cand)
        flag = auxbuf[pl.ds(AUX_FLAGS + row, L)][0]
        y = jnp.where(flag != 0, p, jnp.int32(END_TOKEN_VAL))
        ybuf[...] = jnp.full((L,), y, jnp.int32)
        pltpu.sync_copy(ybuf, y_hbm.at[row])

    return k(hid, aux)


def _tc_body(upi_ref, hid_ref, flags_ref, y_ref, macc_ref, iacc_ref):
    j = pl.program_id(0)
    x = hid_ref[...].reshape(BTC, TBLK)
    colidx = (j * TBLK
              + jax.lax.broadcasted_iota(jnp.int32, (BTC, TBLK), 1))
    x = jnp.where(colidx < V, x, -jnp.inf)

    @pl.when(j == 0)
    def _():
        macc_ref[...] = jnp.full((BTC, 128), -jnp.inf, jnp.float32)
        iacc_ref[...] = jnp.zeros((BTC, 128), jnp.int32)

    mac = macc_ref[...]
    iac = iacc_ref[...]
    for k in range(TBLK // 128):
        v = x[:, k * 128:(k + 1) * 128]
        ci = colidx[:, k * 128:(k + 1) * 128]
        upd = v > mac
        mac = jnp.where(upd, v, mac)
        iac = jnp.where(upd, ci, iac)
    macc_ref[...] = mac
    iacc_ref[...] = iac

    @pl.when(j == TNB - 1)
    def _():
        rowmax = jnp.max(mac, axis=1, keepdims=True)
        cand = jnp.where(mac == rowmax, iac, BIG)
        p = jnp.min(cand, axis=1, keepdims=True)
        flag = flags_ref[...]
        y = jnp.where(flag != 0, p, END_TOKEN_VAL).astype(jnp.int32)
        y_ref[...] = jnp.broadcast_to(y, (BTC, 128))


def _greedy_tc(hid, upi_arr, flags_tc):
    grid_spec = pltpu.PrefetchScalarGridSpec(
        num_scalar_prefetch=1,
        grid=(TNB,),
        in_specs=[
            pl.BlockSpec((BTC, 1, TBLK), lambda j, upi: (1, 0, j)),
            pl.BlockSpec((BTC, 1), lambda j, upi: (0, 0)),
        ],
        out_specs=pl.BlockSpec((BTC, 128), lambda j, upi: (0, 0)),
        scratch_shapes=[
            pltpu.VMEM((BTC, 128), jnp.float32),
            pltpu.VMEM((BTC, 128), jnp.int32),
        ],
    )
    return pl.pallas_call(
        _tc_body,
        grid_spec=grid_spec,
        out_shape=jax.ShapeDtypeStruct((BTC, 128), jnp.int32),
    )(upi_arr[:1], hid, flags_tc)


def _assemble_body(upi_ref, outids_ref, y_ref, out_ref):
    upi = upi_ref[0]
    cols = jax.lax.broadcasted_iota(jnp.int32, (B, S), 1)
    out_ref[...] = jnp.where(cols == upi, y_ref[...], outids_ref[...])


def _assemble_tc(upi_arr, out_ids, y_all):
    grid_spec = pltpu.PrefetchScalarGridSpec(
        num_scalar_prefetch=1,
        grid=(1,),
        in_specs=[
            pl.BlockSpec((B, S), lambda j, upi: (0, 0)),
            pl.BlockSpec((B, 1), lambda j, upi: (0, 0)),
        ],
        out_specs=pl.BlockSpec((B, S), lambda j, upi: (0, 0)),
    )
    return pl.pallas_call(
        _assemble_body,
        grid_spec=grid_spec,
        out_shape=jax.ShapeDtypeStruct((B, S), jnp.int32),
    )(upi_arr[:1], out_ids, y_all)


def kernel(hidden_state, update_index, out_ids, flags):
    flags_flat = flags.reshape(B).astype(jnp.int32)
    upi_arr = jnp.full((16,), update_index, jnp.int32)
    tails_i32 = jax.lax.bitcast_convert_type(
        hidden_state[:BSC, 0, V - VT:], jnp.int32).reshape(BSC * VT)
    aux = jnp.zeros((AUX_LEN,), jnp.int32)
    aux = aux.at[:16].set(upi_arr)
    aux = aux.at[AUX_FLAGS:AUX_FLAGS + BSC].set(flags_flat[:BSC])
    aux = aux.at[AUX_TAILS:].set(tails_i32)

    y_sc = _greedy_sc(hidden_state, aux)
    y_tc = _greedy_tc(hidden_state, upi_arr,
                      flags_flat[BSC:].reshape(BTC, 1))
    y_all = jnp.concatenate([y_sc[:, :1], y_tc[:, :1]], axis=0)
    out = _assemble_tc(upi_arr, out_ids, y_all)
    new_flags = y_all != END_TOKEN_VAL
    return out, new_flags
